# baseline (speedup 1.0000x reference)
import jax
import jax.numpy as jnp
from jax import lax
from jax.experimental import pallas as pl

N = 16384
D = 128
BLOCK = 4096


def _vq_kernel(x_ref, m_ref, r_ref, bc_ref, rc_ref, o_ref):
    xb = x_ref[:]
    norm = jnp.sqrt(jnp.sum(xb * xb, axis=-1, keepdims=True))
    xn = xb / (norm + 1e-8)
    r = r_ref[:]
    xr = lax.dot_general(xn, r, (((1,), (0,)), ((), ())),
                         preferred_element_type=jnp.float32)

    c = jnp.concatenate([bc_ref[:], rc_ref[:],
                         jnp.zeros((6, D), jnp.float32)], axis=0)
    st = lax.dot_general(c, xr, (((1,), (1,)), ((), ())),
                         preferred_element_type=jnp.float32)

    base_idx = (st[1:2, :] > st[0:1, :]).astype(jnp.int32)
    rmax = st[2:3, :]
    for k in range(3, 10):
        rmax = jnp.maximum(rmax, st[k:k + 1, :])
    refine_idx = jnp.full_like(base_idx, 9)
    for k in range(8, 1, -1):
        refine_idx = jnp.where(st[k:k + 1, :] == rmax, k, refine_idx)

    mask = m_ref[:]
    sel = jnp.where(mask, refine_idx, base_idx)
    B = xb.shape[0]
    row = lax.broadcasted_iota(jnp.int32, (48, B), 0)
    oh3 = jnp.where((row == sel) | (row == sel + 16) | (row == sel + 32),
                    1.0, 0.0)

    c_hi = c.astype(jnp.bfloat16).astype(jnp.float32)
    c_mid = (c - c_hi).astype(jnp.bfloat16).astype(jnp.float32)
    c_lo = c - c_hi - c_mid
    cs = jnp.concatenate([c_hi, c_mid, c_lo], axis=0)
    recon = lax.dot_general(oh3, cs, (((0,), (0,)), ((), ())),
                            preferred_element_type=jnp.float32)

    resid = xr - recon
    ones_s = jnp.full((D, 8), 1.0 / D, jnp.float32)
    scale = lax.dot_general(jnp.abs(resid), ones_s, (((1,), (0,)), ((), ())),
                            preferred_element_type=jnp.float32)[:, 0:1]
    full_rot = recon + jnp.where(resid >= 0, scale, -scale)

    out = lax.dot_general(full_rot, r, (((1,), (1,)), ((), ())),
                          preferred_element_type=jnp.float32)
    o_ref[:] = out * norm


@jax.jit
def kernel(x, refine_mask, R, base_centroids, refine_centroids):
    m = refine_mask.reshape(1, N)
    grid = (N // BLOCK,)
    return pl.pallas_call(
        _vq_kernel,
        grid=grid,
        in_specs=[
            pl.BlockSpec((BLOCK, D), lambda i: (i, 0)),
            pl.BlockSpec((1, BLOCK), lambda i: (0, i)),
            pl.BlockSpec((D, D), lambda i: (0, 0)),
            pl.BlockSpec((2, D), lambda i: (0, 0)),
            pl.BlockSpec((8, D), lambda i: (0, 0)),
        ],
        out_specs=pl.BlockSpec((BLOCK, D), lambda i: (i, 0)),
        out_shape=jax.ShapeDtypeStruct((N, D), jnp.float32),
    )(x, m, R, base_centroids, refine_centroids)


# speedup vs baseline: 1.1843x; 1.1843x over previous
import jax
import jax.numpy as jnp
from jax import lax
from jax.experimental import pallas as pl

N = 16384
D = 128
BLOCK = 4096


def _vq_kernel(x_ref, m_ref, r_ref, bc_ref, rc_ref, o_ref):
    xb = x_ref[:]
    norm = jnp.sqrt(jnp.sum(xb * xb, axis=-1, keepdims=True))
    xn = xb / (norm + 1e-8)
    r = r_ref[:]
    xr = lax.dot_general(xn, r, (((1,), (0,)), ((), ())),
                         preferred_element_type=jnp.float32)

    c = jnp.concatenate([bc_ref[:], rc_ref[:],
                         jnp.zeros((6, D), jnp.float32)], axis=0)
    st = lax.dot_general(c, xr, (((1,), (1,)), ((), ())),
                         preferred_element_type=jnp.float32)

    base_idx = (st[1:2, :] > st[0:1, :]).astype(jnp.int32)
    rmax = st[2:3, :]
    for k in range(3, 10):
        rmax = jnp.maximum(rmax, st[k:k + 1, :])
    refine_idx = jnp.full_like(base_idx, 9)
    for k in range(8, 1, -1):
        refine_idx = jnp.where(st[k:k + 1, :] == rmax, k, refine_idx)

    mask = m_ref[:]
    sel = jnp.where(mask, refine_idx, base_idx)
    B = xb.shape[0]
    row = lax.broadcasted_iota(jnp.int32, (16, B), 0)
    oht = (row == sel).astype(jnp.float32)
    oh3 = jnp.concatenate([oht, oht, oht], axis=0)

    c_hi = c.astype(jnp.bfloat16).astype(jnp.float32)
    c_mid = (c - c_hi).astype(jnp.bfloat16).astype(jnp.float32)
    c_lo = c - c_hi - c_mid
    cs = jnp.concatenate([c_hi, c_mid, c_lo], axis=0)
    recon = lax.dot_general(oh3, cs, (((0,), (0,)), ((), ())),
                            preferred_element_type=jnp.float32)

    resid = xr - recon
    scale = jnp.mean(jnp.abs(resid), axis=-1, keepdims=True)
    full_rot = recon + jnp.where(resid >= 0, scale, -scale)

    out = lax.dot_general(full_rot, r, (((1,), (1,)), ((), ())),
                          preferred_element_type=jnp.float32)
    o_ref[:] = out * norm


@jax.jit
def kernel(x, refine_mask, R, base_centroids, refine_centroids):
    m = refine_mask.reshape(1, N)
    grid = (N // BLOCK,)
    return pl.pallas_call(
        _vq_kernel,
        grid=grid,
        in_specs=[
            pl.BlockSpec((BLOCK, D), lambda i: (i, 0)),
            pl.BlockSpec((1, BLOCK), lambda i: (0, i)),
            pl.BlockSpec((D, D), lambda i: (0, 0)),
            pl.BlockSpec((2, D), lambda i: (0, 0)),
            pl.BlockSpec((8, D), lambda i: (0, 0)),
        ],
        out_specs=pl.BlockSpec((BLOCK, D), lambda i: (i, 0)),
        out_shape=jax.ShapeDtypeStruct((N, D), jnp.float32),
    )(x, m, R, base_centroids, refine_centroids)
